# Initial kernel scaffold; baseline (speedup 1.0000x reference)
import jax
import jax.numpy as jnp
from jax.experimental import pallas as pl


def kernel(v):
    raise NotImplementedError("write your pallas kernel here")


# speedup vs baseline: 26.8128x; 26.8128x over previous
import numpy as np
import jax
import jax.numpy as jnp
from jax.experimental import pallas as pl
from jax.experimental.pallas import tpu as pltpu

_M = 262144
_ALPHA = 0.1
_REG = 0.01
_TOL = 1e-4
_CUTOFF = int(_ALPHA * _M)
_SURPLUS = 1.0 - _CUTOFF / (_ALPHA * _M)
_LOG_M = float(np.log(_M))
_INV_AM = 1.0 / (_ALPHA * _M)
_KL_HARD = _LOG_M + _CUTOFF * _INV_AM * np.log(_INV_AM) + _SURPLUS * np.log(_SURPLUS)
_LOG_INV_ALPHA = float(np.log(1.0 / _ALPHA))
_IMIN = np.int32(-(2**31))
_XMASK = np.int32(0x7FFFFFFF)


def _unkey(k):
    bb = jnp.where(k >= 0, k, k ^ _XMASK)
    return jax.lax.bitcast_convert_type(bb, jnp.float32)


def _body(v_ref, out_ref, keys_ref):
    v = v_ref[...]
    b = jax.lax.bitcast_convert_type(v, jnp.int32)
    keys_ref[...] = jnp.where(b >= 0, b, b ^ _XMASK)

    vmax = jnp.max(v)
    e = jnp.exp((v - vmax) * (1.0 / _REG))
    S = jnp.sum(e)
    w = jnp.minimum(e * (_M / S), 1.0 / _ALPHA)
    target = 1.0 - jnp.sum(w) * (1.0 / _M)
    dot_soft = jnp.sum(w * v) * (1.0 / _M)
    x = (v - vmax) * (1.0 / _REG) - jnp.log(S) + _LOG_M
    ent = jnp.sum(w * (jnp.minimum(x, _LOG_INV_ALPHA) - _LOG_M)) * (1.0 / _M)
    soft_val = dot_soft - _REG * (_LOG_M + ent)

    c_sign = jnp.sum((keys_ref[...] >= 0).astype(jnp.int32))
    t0 = jnp.where(c_sign >= _CUTOFF, jnp.int32(0), _IMIN)

    def step(i, t):
        cand = t + (jnp.int32(1) << (30 - i))
        c = jnp.sum((keys_ref[...] >= cand).astype(jnp.int32))
        return jnp.where(c >= _CUTOFF, cand, t)

    t1k = jax.lax.fori_loop(0, 31, step, t0)
    keys = keys_ref[...]
    c_gt = jnp.sum((keys > t1k).astype(jnp.int32))
    c_ge = jnp.sum((keys >= t1k).astype(jnp.int32))
    s_gt = jnp.sum(jnp.where(keys > t1k, v, 0.0))
    t2k = jnp.where(c_ge >= _CUTOFF + 1, t1k,
                    jnp.max(jnp.where(keys < t1k, keys, _IMIN)))
    t1 = _unkey(t1k)
    t2 = _unkey(t2k)
    s_top = s_gt + (jnp.float32(_CUTOFF) - c_gt.astype(jnp.float32)) * t1
    hard_val = s_top * _INV_AM + _SURPLUS * t2 - _REG * _KL_HARD

    res = jnp.where(jnp.abs(target) <= _TOL, soft_val, hard_val)
    out_ref[...] = jnp.broadcast_to(res, (1, 1))


def kernel(v):
    vv = v.reshape(2048, 128)
    out = pl.pallas_call(
        _body,
        out_shape=jax.ShapeDtypeStruct((1, 1), jnp.float32),
        scratch_shapes=[pltpu.VMEM((2048, 128), jnp.int32)],
    )(vv)
    return out[0, 0]
